# baseline (speedup 1.0000x reference)
import functools

import jax
import jax.numpy as jnp
from jax import lax
from jax.experimental import pallas as pl
from jax.experimental.pallas import tpu as pltpu
from jax.experimental.pallas import tpu_sc as plsc

HIDDEN = 768
NVEC = HIDDEN // 16
NT = 8192
NW = 32
TPW = NT // NW
CH = 32
NCH = TPW // CH
NBUF = 4
GRP = 4
NGRP = CH // GRP


def _take(v, idx):
    dnums = lax.GatherDimensionNumbers(
        offset_dims=(), collapsed_slice_dims=(0,), start_index_map=(0,))
    return lax.gather(v, idx[:, None], dnums, slice_sizes=(1,),
                      mode=lax.GatherScatterMode.PROMISE_IN_BOUNDS)


def _splat(v, j):
    return _take(v, jnp.full((16,), j, jnp.int32))


def _fold16(vs, lanes):
    sh = 1
    while len(vs) > 1:
        mask = (lanes & sh) == 0
        nxt = []
        for m in range(0, len(vs), 2):
            a, b = vs[m], vs[m + 1]
            pa = a + _take(a, lanes ^ sh)
            pb = b + _take(b, lanes ^ sh)
            nxt.append(jnp.where(mask, pa, pb))
        vs = nxt
        sh *= 2
    v = vs[0]
    while sh < 16:
        v = v + _take(v, lanes ^ sh)
        sh *= 2
    return v


def _rsqrt_newton(v):
    i = lax.bitcast_convert_type(v, jnp.int32)
    i = jnp.int32(0x5F3759DF) - lax.shift_right_arithmetic(i, 1)
    y = lax.bitcast_convert_type(i, jnp.float32)
    for _ in range(3):
        y = y * (jnp.float32(1.5) - jnp.float32(0.5) * v * y * y)
    return y


def _chunk_compute(rows_v, tidf_v, toff, t0_v, t1_v, g_v, b_v, lanes):
    zero = jnp.zeros((16,), jnp.float32)

    def group_body(g, carry):
        gbase = g * GRP
        tv = tidf_v[pl.ds(toff + gbase, GRP)]
        tfs = [_splat(tv, j) for j in range(GRP)]

        def p1(i, sq):
            s, q = sq
            sl = pl.ds(16 * i, 16)
            t0i = t0_v[sl]
            di = t1_v[sl] - t0i
            ns, nq = [], []
            for j in range(GRP):
                e = rows_v[gbase + j, sl] + (t0i + tfs[j] * di)
                rows_v[gbase + j, sl] = e
                ns.append(s[j] + e)
                nq.append(q[j] + e * e)
            return (tuple(ns), tuple(nq))

        s, q = plsc.parallel_loop(
            0, NVEC, carry=((zero,) * GRP, (zero,) * GRP), unroll=2)(p1)

        tot_s = _fold16(list(s), lanes)
        tot_q = _fold16(list(q), lanes)

        mean = tot_s * jnp.float32(1.0 / HIDDEN)
        var = jnp.maximum(tot_q * jnp.float32(1.0 / HIDDEN) - mean * mean,
                          jnp.float32(0.0)) + jnp.float32(1e-5)
        rs = _rsqrt_newton(var)
        a = [_splat(rs, j) for j in range(GRP)]
        mu = [_splat(mean, j) for j in range(GRP)]

        def p2(i, c):
            sl = pl.ds(16 * i, 16)
            gi = g_v[sl]
            bi = b_v[sl]
            for j in range(GRP):
                e = rows_v[gbase + j, sl]
                rows_v[gbase + j, sl] = (e - mu[j]) * a[j] * gi + bi
            return c

        plsc.parallel_loop(0, NVEC, carry=jnp.int32(0), unroll=2)(p2)
        return carry

    lax.fori_loop(0, NGRP, group_body, 0)


def _body(tok_hbm, tidf_hbm, table_hbm, t0_hbm, t1_hbm, g_hbm, b_hbm, out_hbm,
          *refs):
    idx_v, tidf_v = refs[0:2]
    rows = refs[2:2 + NBUF]
    t0_v, t1_v, g_v, b_v = refs[2 + NBUF:6 + NBUF]
    gsems = refs[6 + NBUF:6 + 2 * NBUF]
    wsems = refs[6 + 2 * NBUF:6 + 3 * NBUF]

    cid = lax.axis_index("c")
    sid = lax.axis_index("s")
    wid = sid * 2 + cid
    base = wid * TPW

    lanes = lax.broadcasted_iota(jnp.int32, (16,), 0)

    pltpu.sync_copy(tok_hbm.at[pl.ds(base, TPW)], idx_v)
    pltpu.sync_copy(tidf_hbm.at[pl.ds(base, TPW)], tidf_v)

    def stage(c):
        p = c % NBUF
        return pltpu.async_copy(table_hbm.at[idx_v.at[pl.ds(c * CH, CH)]],
                                rows[p], gsems[p])

    gh = [None] * NCH
    wh = [None] * NCH
    gh[0] = stage(0)
    gh[1] = stage(1)

    pltpu.sync_copy(t0_hbm, t0_v)
    pltpu.sync_copy(t1_hbm, t1_v)
    pltpu.sync_copy(g_hbm, g_v)
    pltpu.sync_copy(b_hbm, b_v)

    for c in range(NCH):
        p = c % NBUF
        gh[c].wait()
        if c + 2 < NCH:
            if c - 2 >= 0:
                wh[c - 2].wait()
            gh[c + 2] = stage(c + 2)
        _chunk_compute(rows[p], tidf_v, c * CH, t0_v, t1_v, g_v, b_v, lanes)
        wh[c] = pltpu.async_copy(rows[p], out_hbm.at[pl.ds(base + c * CH, CH)],
                                 wsems[p])
    for c in range(max(0, NCH - 4), NCH):
        wh[c].wait()


@functools.cache
def _build():
    mesh = plsc.VectorSubcoreMesh(core_axis_name="c", subcore_axis_name="s")
    return pl.kernel(
        _body,
        out_type=jax.ShapeDtypeStruct((NT, HIDDEN), jnp.float32),
        mesh=mesh,
        scratch_types=(
            [pltpu.VMEM((TPW,), jnp.int32), pltpu.VMEM((TPW,), jnp.float32)]
            + [pltpu.VMEM((CH, HIDDEN), jnp.float32) for _ in range(NBUF)]
            + [pltpu.VMEM((HIDDEN,), jnp.float32) for _ in range(4)]
            + [pltpu.SemaphoreType.DMA for _ in range(2 * NBUF)]
        ),
    )


def kernel(token_ids, type_ids, token_table, type_table, ln_gamma, ln_beta):
    tok = token_ids.reshape(-1).astype(jnp.int32)
    tidf = type_ids.reshape(-1).astype(jnp.float32)
    t0 = type_table[0]
    t1 = type_table[1]
    out = _build()(tok, tidf, token_table, t0, t1, ln_gamma, ln_beta)
    return out.reshape(*token_ids.shape, HIDDEN)


# speedup vs baseline: 1.2442x; 1.2442x over previous
import functools

import jax
import jax.numpy as jnp
from jax import lax
from jax.experimental import pallas as pl
from jax.experimental.pallas import tpu as pltpu
from jax.experimental.pallas import tpu_sc as plsc

HIDDEN = 768
NVEC = HIDDEN // 16
NT = 8192
NW = 32
TPW = NT // NW
CH = 32
NCH = TPW // CH
NBUF = 4
GRP = 8
NGRP = CH // GRP


def _take(v, idx):
    dnums = lax.GatherDimensionNumbers(
        offset_dims=(), collapsed_slice_dims=(0,), start_index_map=(0,))
    return lax.gather(v, idx[:, None], dnums, slice_sizes=(1,),
                      mode=lax.GatherScatterMode.PROMISE_IN_BOUNDS)


def _splat(v, j):
    return _take(v, jnp.full((16,), j, jnp.int32))


def _fold16(vs, lanes):
    sh = 1
    while len(vs) > 1:
        mask = (lanes & sh) == 0
        nxt = []
        for m in range(0, len(vs), 2):
            a, b = vs[m], vs[m + 1]
            pa = a + _take(a, lanes ^ sh)
            pb = b + _take(b, lanes ^ sh)
            nxt.append(jnp.where(mask, pa, pb))
        vs = nxt
        sh *= 2
    v = vs[0]
    while sh < 16:
        v = v + _take(v, lanes ^ sh)
        sh *= 2
    return v


def _rsqrt_newton(v):
    i = lax.bitcast_convert_type(v, jnp.int32)
    i = jnp.int32(0x5F3759DF) - lax.shift_right_arithmetic(i, 1)
    y = lax.bitcast_convert_type(i, jnp.float32)
    for _ in range(3):
        y = y * (jnp.float32(1.5) - jnp.float32(0.5) * v * y * y)
    return y


def _chunk_compute(rows_v, tidf_v, toff, t0_v, t1_v, g_v, b_v, lanes):
    zero = jnp.zeros((16,), jnp.float32)

    def group_body(g, carry):
        gbase = g * GRP
        tv = tidf_v[pl.ds(toff + gbase, GRP)]
        tfs = [_splat(tv, j) for j in range(GRP)]

        def p1(i, sq):
            s, q = sq
            sl = pl.ds(16 * i, 16)
            t0i = t0_v[sl]
            di = t1_v[sl] - t0i
            ns, nq = [], []
            for j in range(GRP):
                e = rows_v[gbase + j, sl] + (t0i + tfs[j] * di)
                rows_v[gbase + j, sl] = e
                ns.append(s[j] + e)
                nq.append(q[j] + e * e)
            return (tuple(ns), tuple(nq))

        s, q = plsc.parallel_loop(
            0, NVEC, carry=((zero,) * GRP, (zero,) * GRP), unroll=2)(p1)

        tot_s = _fold16(list(s), lanes)
        tot_q = _fold16(list(q), lanes)

        mean = tot_s * jnp.float32(1.0 / HIDDEN)
        var = jnp.maximum(tot_q * jnp.float32(1.0 / HIDDEN) - mean * mean,
                          jnp.float32(0.0)) + jnp.float32(1e-5)
        rs = _rsqrt_newton(var)
        a = [_splat(rs, j) for j in range(GRP)]
        mu = [_splat(mean, j) for j in range(GRP)]

        def p2(i, c):
            sl = pl.ds(16 * i, 16)
            gi = g_v[sl]
            bi = b_v[sl]
            for j in range(GRP):
                e = rows_v[gbase + j, sl]
                rows_v[gbase + j, sl] = (e - mu[j]) * a[j] * gi + bi
            return c

        plsc.parallel_loop(0, NVEC, carry=jnp.int32(0), unroll=2)(p2)
        return carry

    lax.fori_loop(0, NGRP, group_body, 0)


def _body(tok_hbm, tidf_hbm, table_hbm, t0_hbm, t1_hbm, g_hbm, b_hbm, out_hbm,
          *refs):
    idx_v, tidf_v = refs[0:2]
    rows = refs[2:2 + NBUF]
    t0_v, t1_v, g_v, b_v = refs[2 + NBUF:6 + NBUF]
    gsems = refs[6 + NBUF:6 + 2 * NBUF]
    wsems = refs[6 + 2 * NBUF:6 + 3 * NBUF]

    cid = lax.axis_index("c")
    sid = lax.axis_index("s")
    wid = sid * 2 + cid
    base = wid * TPW

    lanes = lax.broadcasted_iota(jnp.int32, (16,), 0)

    pltpu.sync_copy(tok_hbm.at[pl.ds(base, TPW)], idx_v)
    pltpu.sync_copy(tidf_hbm.at[pl.ds(base, TPW)], tidf_v)

    def stage(c):
        p = c % NBUF
        return pltpu.async_copy(table_hbm.at[idx_v.at[pl.ds(c * CH, CH)]],
                                rows[p], gsems[p])

    gh = [None] * NCH
    wh = [None] * NCH
    gh[0] = stage(0)
    gh[1] = stage(1)

    pltpu.sync_copy(t0_hbm, t0_v)
    pltpu.sync_copy(t1_hbm, t1_v)
    pltpu.sync_copy(g_hbm, g_v)
    pltpu.sync_copy(b_hbm, b_v)

    for c in range(NCH):
        p = c % NBUF
        gh[c].wait()
        if c + 2 < NCH:
            if c - 2 >= 0:
                wh[c - 2].wait()
            gh[c + 2] = stage(c + 2)
        _chunk_compute(rows[p], tidf_v, c * CH, t0_v, t1_v, g_v, b_v, lanes)
        wh[c] = pltpu.async_copy(rows[p], out_hbm.at[pl.ds(base + c * CH, CH)],
                                 wsems[p])
    for c in range(max(0, NCH - 4), NCH):
        wh[c].wait()


@functools.cache
def _build():
    mesh = plsc.VectorSubcoreMesh(core_axis_name="c", subcore_axis_name="s")
    return pl.kernel(
        _body,
        out_type=jax.ShapeDtypeStruct((NT, HIDDEN), jnp.float32),
        mesh=mesh,
        scratch_types=(
            [pltpu.VMEM((TPW,), jnp.int32), pltpu.VMEM((TPW,), jnp.float32)]
            + [pltpu.VMEM((CH, HIDDEN), jnp.float32) for _ in range(NBUF)]
            + [pltpu.VMEM((HIDDEN,), jnp.float32) for _ in range(4)]
            + [pltpu.SemaphoreType.DMA for _ in range(2 * NBUF)]
        ),
    )


def kernel(token_ids, type_ids, token_table, type_table, ln_gamma, ln_beta):
    tok = token_ids.reshape(-1).astype(jnp.int32)
    tidf = type_ids.reshape(-1).astype(jnp.float32)
    t0 = type_table[0]
    t1 = type_table[1]
    out = _build()(tok, tidf, token_table, t0, t1, ln_gamma, ln_beta)
    return out.reshape(*token_ids.shape, HIDDEN)
